# Initial kernel scaffold; baseline (speedup 1.0000x reference)
import jax
import jax.numpy as jnp
from jax.experimental import pallas as pl


def kernel(feat, img_map, kg_map, all_map, node_id, pre_node_id, img_feat, kg_feat, W1, b1, W2, b2):
    raise NotImplementedError("write your pallas kernel here")


# speedup vs baseline: 1.8514x; 1.8514x over previous
import functools

import jax
import jax.numpy as jnp
from jax import lax
from jax.experimental import pallas as pl
from jax.experimental.pallas import tpu as pltpu
from jax.experimental.pallas import tpu_sc as plsc

N = 50000
D = 128
M = 256
B = 8192
MID = 256

NC = 2
NS = 16
NW = NC * NS
BPW = B // NW
CH = 128
NCH = BPW // CH

BLK = 256
NBLK = B // BLK

_f32 = jnp.float32
_i32 = jnp.int32

_sc_mesh = plsc.VectorSubcoreMesh(
    core_axis_name="c", subcore_axis_name="s", num_cores=NC, num_subcores=NS)


def _sc_gather_body(feat_hbm, img_hbm, kg_hbm, nid_hbm, pid_hbm,
                    out_ch, out_ci, out_ck, out_pi, out_pk,
                    nid_v, pid_v, fbuf, mbuf, sem):
  wid = lax.axis_index("s") * NC + lax.axis_index("c")
  base = wid * BPW
  pltpu.sync_copy(nid_hbm.at[pl.ds(wid * NCH, NCH)], nid_v)
  pltpu.sync_copy(pid_hbm.at[pl.ds(wid * NCH, NCH)], pid_v)
  for j in range(NCH):
    o = base + j * CH
    pltpu.async_copy(feat_hbm.at[nid_v.at[j]], fbuf, sem).wait()
    pltpu.sync_copy(fbuf, out_ch.at[pl.ds(o, CH)])
    pltpu.async_copy(img_hbm.at[nid_v.at[j]], mbuf, sem).wait()
    pltpu.sync_copy(mbuf, out_ci.at[pl.ds(o, CH)])
    pltpu.async_copy(kg_hbm.at[nid_v.at[j]], mbuf, sem).wait()
    pltpu.sync_copy(mbuf, out_ck.at[pl.ds(o, CH)])
    pltpu.async_copy(img_hbm.at[pid_v.at[j]], mbuf, sem).wait()
    pltpu.sync_copy(mbuf, out_pi.at[pl.ds(o, CH)])
    pltpu.async_copy(kg_hbm.at[pid_v.at[j]], mbuf, sem).wait()
    pltpu.sync_copy(mbuf, out_pk.at[pl.ds(o, CH)])


_sc_gather = pl.kernel(
    _sc_gather_body,
    out_type=[
        jax.ShapeDtypeStruct((B, D), _f32),
        jax.ShapeDtypeStruct((B, M), _f32),
        jax.ShapeDtypeStruct((B, M), _f32),
        jax.ShapeDtypeStruct((B, M), _f32),
        jax.ShapeDtypeStruct((B, M), _f32),
    ],
    mesh=_sc_mesh,
    scratch_types=[
        pltpu.VMEM((NCH, CH), _i32),
        pltpu.VMEM((NCH, CH), _i32),
        pltpu.VMEM((CH, D), _f32),
        pltpu.VMEM((CH, M), _f32),
        pltpu.SemaphoreType.DMA,
    ],
)


def _dense_body(nid_row3_ref, nid_colb_ref, ch_ref, ci_ref, ck_ref,
                pi_ref, pk_ref, w1_ref, b1_ref, w2_ref, b2_ref,
                if_ref, kf_ref,
                ni_ref, nk_ref, na_ref, nf_ref, win_ref):
  hp = dict(preferred_element_type=_f32, precision=lax.Precision.HIGHEST)
  h = jnp.maximum(jnp.dot(ch_ref[...], w1_ref[...], **hp) + b1_ref[...], 0.0)
  glin = jnp.sum(h * w2_ref[...], axis=1, keepdims=True) + b2_ref[0:1, 0:1]
  g = 1.0 / (1.0 + jnp.exp(-glin))
  ni = ci_ref[...] * g + (1.0 - g) * pi_ref[...]
  nk = ck_ref[...] * g + (1.0 - g) * pk_ref[...]
  m = jnp.maximum(jnp.max(ni, axis=1, keepdims=True),
                  jnp.max(nk, axis=1, keepdims=True))
  ei = jnp.exp(ni - m)
  ek = jnp.exp(nk - m)
  s = jnp.sum(ei, axis=1, keepdims=True) + jnp.sum(ek, axis=1, keepdims=True)
  ai = ei / s
  ak = ek / s
  ni_ref[...] = ni
  nk_ref[...] = nk
  na_ref[...] = jnp.concatenate([ai, ak], axis=1)
  nf_ref[...] = (jnp.dot(ai, if_ref[...], **hp) +
                 jnp.dot(ak, kf_ref[...], **hp))
  ids_mine = nid_row3_ref[0]
  winr = jnp.full((1, BLK), -1, _i32)
  for t in range(B // BLK):
    colc = nid_colb_ref[t * BLK:(t + 1) * BLK, 0:1]
    eq = colc == ids_mine
    jcol = lax.broadcasted_iota(_i32, (BLK, BLK), 0) + t * BLK
    winr = jnp.maximum(
        winr, jnp.max(jnp.where(eq, jcol, -1), axis=0, keepdims=True))
  win_ref[0] = winr


_dense = pl.pallas_call(
    _dense_body,
    grid=(NBLK,),
    in_specs=[
        pl.BlockSpec((1, 1, BLK), lambda i: (i, 0, 0)),
        pl.BlockSpec((B, 128), lambda i: (0, 0)),
        pl.BlockSpec((BLK, D), lambda i: (i, 0)),
        pl.BlockSpec((BLK, M), lambda i: (i, 0)),
        pl.BlockSpec((BLK, M), lambda i: (i, 0)),
        pl.BlockSpec((BLK, M), lambda i: (i, 0)),
        pl.BlockSpec((BLK, M), lambda i: (i, 0)),
        pl.BlockSpec((D, MID), lambda i: (0, 0)),
        pl.BlockSpec((1, MID), lambda i: (0, 0)),
        pl.BlockSpec((1, MID), lambda i: (0, 0)),
        pl.BlockSpec((1, 128), lambda i: (0, 0)),
        pl.BlockSpec((M, D), lambda i: (0, 0)),
        pl.BlockSpec((M, D), lambda i: (0, 0)),
    ],
    out_specs=[
        pl.BlockSpec((BLK, M), lambda i: (i, 0)),
        pl.BlockSpec((BLK, M), lambda i: (i, 0)),
        pl.BlockSpec((BLK, 2 * M), lambda i: (i, 0)),
        pl.BlockSpec((BLK, D), lambda i: (i, 0)),
        pl.BlockSpec((1, 1, BLK), lambda i: (i, 0, 0)),
    ],
    out_shape=[
        jax.ShapeDtypeStruct((B, M), _f32),
        jax.ShapeDtypeStruct((B, M), _f32),
        jax.ShapeDtypeStruct((B, 2 * M), _f32),
        jax.ShapeDtypeStruct((B, D), _f32),
        jax.ShapeDtypeStruct((NBLK, 1, BLK), _i32),
    ],
)


CROWS = 400
CGRID = N // CROWS


def _copy_body(f_in, i_in, k_in, a_in, f_out, i_out, k_out, a_out):
  f_out[...] = f_in[...]
  i_out[...] = i_in[...]
  k_out[...] = k_in[...]
  a_out[...] = a_in[...]


_copy = pl.pallas_call(
    _copy_body,
    grid=(CGRID,),
    in_specs=[
        pl.BlockSpec((CROWS, D), lambda i: (i, 0)),
        pl.BlockSpec((CROWS, M), lambda i: (i, 0)),
        pl.BlockSpec((CROWS, M), lambda i: (i, 0)),
        pl.BlockSpec((CROWS, 2 * M), lambda i: (i, 0)),
    ],
    out_specs=[
        pl.BlockSpec((CROWS, D), lambda i: (i, 0)),
        pl.BlockSpec((CROWS, M), lambda i: (i, 0)),
        pl.BlockSpec((CROWS, M), lambda i: (i, 0)),
        pl.BlockSpec((CROWS, 2 * M), lambda i: (i, 0)),
    ],
    out_shape=[
        jax.ShapeDtypeStruct((N, D), _f32),
        jax.ShapeDtypeStruct((N, M), _f32),
        jax.ShapeDtypeStruct((N, M), _f32),
        jax.ShapeDtypeStruct((N, 2 * M), _f32),
    ],
)


def _sc_scatter_body(nf_hbm, ni_hbm, nk_hbm, na_hbm, nid_hbm, win_hbm,
                     of_ref, oi_ref, ok_ref, oa_ref,
                     nid_v, win_v, fbuf, mbuf, abuf, sem):
  wid = lax.axis_index("s") * NC + lax.axis_index("c")
  pltpu.sync_copy(nid_hbm.at[pl.ds(wid * NCH, NCH)], nid_v)
  pltpu.sync_copy(win_hbm.at[pl.ds(wid * NCH, NCH)], win_v)
  for j in range(NCH):
    pltpu.async_copy(nf_hbm.at[win_v.at[j]], fbuf, sem).wait()
    pltpu.async_copy(fbuf, of_ref.at[nid_v.at[j]], sem).wait()
    pltpu.async_copy(ni_hbm.at[win_v.at[j]], mbuf, sem).wait()
    pltpu.async_copy(mbuf, oi_ref.at[nid_v.at[j]], sem).wait()
    pltpu.async_copy(nk_hbm.at[win_v.at[j]], mbuf, sem).wait()
    pltpu.async_copy(mbuf, ok_ref.at[nid_v.at[j]], sem).wait()
    pltpu.async_copy(na_hbm.at[win_v.at[j]], abuf, sem).wait()
    pltpu.async_copy(abuf, oa_ref.at[nid_v.at[j]], sem).wait()


_sc_scatter = pl.kernel(
    _sc_scatter_body,
    out_type=(),
    mesh=_sc_mesh,
    scratch_types=[
        pltpu.VMEM((NCH, CH), _i32),
        pltpu.VMEM((NCH, CH), _i32),
        pltpu.VMEM((CH, D), _f32),
        pltpu.VMEM((CH, M), _f32),
        pltpu.VMEM((CH, 2 * M), _f32),
        pltpu.SemaphoreType.DMA,
    ],
)


def kernel(feat, img_map, kg_map, all_map, node_id, pre_node_id,
           img_feat, kg_feat, W1, b1, W2, b2):
  nid = node_id.astype(_i32)
  pid = pre_node_id.astype(_i32)
  nid2 = nid.reshape(NW * NCH, CH)
  pid2 = pid.reshape(NW * NCH, CH)

  cur_h, cur_img, cur_kg, pre_img, pre_kg = _sc_gather(
      feat, img_map, kg_map, nid2, pid2)

  nid_row3 = nid.reshape(NBLK, 1, BLK)
  nid_colb = jnp.broadcast_to(nid.reshape(B, 1), (B, 128))
  b1r = b1.reshape(1, MID)
  w2r = W2.reshape(1, MID)
  b2r = jnp.broadcast_to(b2.reshape(1, 1), (1, 128))

  new_img, new_kg, new_all, new_feat, win = _dense(
      nid_row3, nid_colb, cur_h, cur_img, cur_kg, pre_img, pre_kg,
      W1, b1r, w2r, b2r, img_feat, kg_feat)

  out_feat, out_img, out_kg, out_all = _copy(feat, img_map, kg_map, all_map)

  of_r = jax.new_ref(out_feat)
  oi_r = jax.new_ref(out_img)
  ok_r = jax.new_ref(out_kg)
  oa_r = jax.new_ref(out_all)
  win2 = win.reshape(NW * NCH, CH)
  _sc_scatter(new_feat, new_img, new_kg, new_all, nid2, win2,
              of_r, oi_r, ok_r, oa_r)
  return of_r[...], oi_r[...], ok_r[...], oa_r[...]


# speedup vs baseline: 2.0018x; 1.0813x over previous
import functools

import jax
import jax.numpy as jnp
from jax import lax
from jax.experimental import pallas as pl
from jax.experimental.pallas import tpu as pltpu
from jax.experimental.pallas import tpu_sc as plsc

N = 50000
D = 128
M = 256
B = 8192
MID = 256

NC = 2
NS = 16
NW = NC * NS
BPW = B // NW
CH = 128
NCH = BPW // CH

BLK = 256
NBLK = B // BLK

_f32 = jnp.float32
_i32 = jnp.int32

_sc_mesh = plsc.VectorSubcoreMesh(
    core_axis_name="c", subcore_axis_name="s", num_cores=NC, num_subcores=NS)


def _sc_gather_body(feat_hbm, img_hbm, kg_hbm, nid_hbm, pid_hbm,
                    out_ch, out_ci, out_ck, out_pi, out_pk,
                    nid_v, pid_v, fbuf, mb0, mb1, mb2,
                    fg, fw, g0, g1, g2, w0, w1, w2):
  wid = lax.axis_index("s") * NC + lax.axis_index("c")
  base = wid * BPW
  pltpu.sync_copy(nid_hbm.at[pl.ds(wid * NCH, NCH)], nid_v)
  pltpu.sync_copy(pid_hbm.at[pl.ds(wid * NCH, NCH)], pid_v)

  mbufs = (mb0, mb1, mb2)
  gsems = (g0, g1, g2)
  wsems = (w0, w1, w2)
  tasks = []
  for j in range(NCH):
    o = base + j * CH
    tasks += [(img_hbm, nid_v, j, out_ci, o),
              (kg_hbm, nid_v, j, out_ck, o),
              (img_hbm, pid_v, j, out_pi, o),
              (kg_hbm, pid_v, j, out_pk, o)]
  nt = len(tasks)
  gcp = [None] * nt
  wcp = [None] * nt

  def fire(t):
    b = t % 3
    if t >= 3:
      wcp[t - 3].wait()
    src, idxr, j, _, _ = tasks[t]
    gcp[t] = pltpu.async_copy(src.at[idxr.at[j]], mbufs[b], gsems[b])

  fg0 = pltpu.async_copy(feat_hbm.at[nid_v.at[0]], fbuf, fg)
  fire(0)
  fire(1)
  fire(2)
  fg0.wait()
  fw_cp = pltpu.async_copy(fbuf, out_ch.at[pl.ds(base, CH)], fw)
  fg1 = None
  for t in range(nt):
    b = t % 3
    gcp[t].wait()
    _, _, _, dst, o = tasks[t]
    wcp[t] = pltpu.async_copy(mbufs[b], dst.at[pl.ds(o, CH)], wsems[b])
    if t == 2:
      fw_cp.wait()
      fg1 = pltpu.async_copy(feat_hbm.at[nid_v.at[1]], fbuf, fg)
    if t + 3 < nt:
      fire(t + 3)
  fg1.wait()
  fw_cp = pltpu.async_copy(fbuf, out_ch.at[pl.ds(base + CH, CH)], fw)
  for t in range(nt - 3, nt):
    wcp[t].wait()
  fw_cp.wait()


_sc_gather = pl.kernel(
    _sc_gather_body,
    out_type=[
        jax.ShapeDtypeStruct((B, D), _f32),
        jax.ShapeDtypeStruct((B, M), _f32),
        jax.ShapeDtypeStruct((B, M), _f32),
        jax.ShapeDtypeStruct((B, M), _f32),
        jax.ShapeDtypeStruct((B, M), _f32),
    ],
    mesh=_sc_mesh,
    scratch_types=[
        pltpu.VMEM((NCH, CH), _i32),
        pltpu.VMEM((NCH, CH), _i32),
        pltpu.VMEM((CH, D), _f32),
        pltpu.VMEM((CH, M), _f32),
        pltpu.VMEM((CH, M), _f32),
        pltpu.VMEM((CH, M), _f32),
        pltpu.SemaphoreType.DMA,
        pltpu.SemaphoreType.DMA,
        pltpu.SemaphoreType.DMA,
        pltpu.SemaphoreType.DMA,
        pltpu.SemaphoreType.DMA,
        pltpu.SemaphoreType.DMA,
        pltpu.SemaphoreType.DMA,
        pltpu.SemaphoreType.DMA,
    ],
)


def _dense_body(nid_row3_ref, nid_colb_ref, ch_ref, ci_ref, ck_ref,
                pi_ref, pk_ref, w1_ref, b1_ref, w2_ref, b2_ref,
                if_ref, kf_ref,
                ni_ref, nk_ref, na_ref, nf_ref, win_ref):
  hp = dict(preferred_element_type=_f32, precision=lax.Precision.HIGHEST)
  h = jnp.maximum(jnp.dot(ch_ref[...], w1_ref[...], **hp) + b1_ref[...], 0.0)
  glin = jnp.sum(h * w2_ref[...], axis=1, keepdims=True) + b2_ref[0:1, 0:1]
  g = 1.0 / (1.0 + jnp.exp(-glin))
  ni = ci_ref[...] * g + (1.0 - g) * pi_ref[...]
  nk = ck_ref[...] * g + (1.0 - g) * pk_ref[...]
  m = jnp.maximum(jnp.max(ni, axis=1, keepdims=True),
                  jnp.max(nk, axis=1, keepdims=True))
  ei = jnp.exp(ni - m)
  ek = jnp.exp(nk - m)
  s = jnp.sum(ei, axis=1, keepdims=True) + jnp.sum(ek, axis=1, keepdims=True)
  ai = ei / s
  ak = ek / s
  ni_ref[...] = ni
  nk_ref[...] = nk
  na_ref[...] = jnp.concatenate([ai, ak], axis=1)
  nf_ref[...] = (jnp.dot(ai, if_ref[...], **hp) +
                 jnp.dot(ak, kf_ref[...], **hp))
  ids_mine = nid_row3_ref[0]
  winr = jnp.full((1, BLK), -1, _i32)
  for t in range(B // BLK):
    colc = nid_colb_ref[t * BLK:(t + 1) * BLK, 0:1]
    eq = colc == ids_mine
    jcol = lax.broadcasted_iota(_i32, (BLK, BLK), 0) + t * BLK
    winr = jnp.maximum(
        winr, jnp.max(jnp.where(eq, jcol, -1), axis=0, keepdims=True))
  win_ref[0] = winr


_dense = pl.pallas_call(
    _dense_body,
    grid=(NBLK,),
    in_specs=[
        pl.BlockSpec((1, 1, BLK), lambda i: (i, 0, 0)),
        pl.BlockSpec((B, 128), lambda i: (0, 0)),
        pl.BlockSpec((BLK, D), lambda i: (i, 0)),
        pl.BlockSpec((BLK, M), lambda i: (i, 0)),
        pl.BlockSpec((BLK, M), lambda i: (i, 0)),
        pl.BlockSpec((BLK, M), lambda i: (i, 0)),
        pl.BlockSpec((BLK, M), lambda i: (i, 0)),
        pl.BlockSpec((D, MID), lambda i: (0, 0)),
        pl.BlockSpec((1, MID), lambda i: (0, 0)),
        pl.BlockSpec((1, MID), lambda i: (0, 0)),
        pl.BlockSpec((1, 128), lambda i: (0, 0)),
        pl.BlockSpec((M, D), lambda i: (0, 0)),
        pl.BlockSpec((M, D), lambda i: (0, 0)),
    ],
    out_specs=[
        pl.BlockSpec((BLK, M), lambda i: (i, 0)),
        pl.BlockSpec((BLK, M), lambda i: (i, 0)),
        pl.BlockSpec((BLK, 2 * M), lambda i: (i, 0)),
        pl.BlockSpec((BLK, D), lambda i: (i, 0)),
        pl.BlockSpec((1, 1, BLK), lambda i: (i, 0, 0)),
    ],
    out_shape=[
        jax.ShapeDtypeStruct((B, M), _f32),
        jax.ShapeDtypeStruct((B, M), _f32),
        jax.ShapeDtypeStruct((B, 2 * M), _f32),
        jax.ShapeDtypeStruct((B, D), _f32),
        jax.ShapeDtypeStruct((NBLK, 1, BLK), _i32),
    ],
)


CROWS = 1000
CGRID = N // CROWS


def _copy_body(f_in, i_in, k_in, a_in, f_out, i_out, k_out, a_out):
  f_out[...] = f_in[...]
  i_out[...] = i_in[...]
  k_out[...] = k_in[...]
  a_out[...] = a_in[...]


_copy = pl.pallas_call(
    _copy_body,
    grid=(CGRID,),
    in_specs=[
        pl.BlockSpec((CROWS, D), lambda i: (i, 0)),
        pl.BlockSpec((CROWS, M), lambda i: (i, 0)),
        pl.BlockSpec((CROWS, M), lambda i: (i, 0)),
        pl.BlockSpec((CROWS, 2 * M), lambda i: (i, 0)),
    ],
    out_specs=[
        pl.BlockSpec((CROWS, D), lambda i: (i, 0)),
        pl.BlockSpec((CROWS, M), lambda i: (i, 0)),
        pl.BlockSpec((CROWS, M), lambda i: (i, 0)),
        pl.BlockSpec((CROWS, 2 * M), lambda i: (i, 0)),
    ],
    out_shape=[
        jax.ShapeDtypeStruct((N, D), _f32),
        jax.ShapeDtypeStruct((N, M), _f32),
        jax.ShapeDtypeStruct((N, M), _f32),
        jax.ShapeDtypeStruct((N, 2 * M), _f32),
    ],
)


CHS = 64
NCHS = BPW // CHS


def _sc_scatter_body(nf_hbm, ni_hbm, nk_hbm, na_hbm, nid_hbm, win_hbm,
                     of_ref, oi_ref, ok_ref, oa_ref,
                     nid_v, win_v, fbuf, mb0, mb1, ab0, ab1,
                     sf, si, sk, sa0, sa1, gf, gi, gk, ga0, ga1):
  wid = lax.axis_index("s") * NC + lax.axis_index("c")
  pltpu.sync_copy(nid_hbm.at[pl.ds(wid * NCHS, NCHS)], nid_v)
  pltpu.sync_copy(win_hbm.at[pl.ds(wid * NCHS, NCHS)], win_v)
  pairs = [
      (nf_hbm, of_ref, (fbuf,), (gf,), (sf,)),
      (ni_hbm, oi_ref, (mb0,), (gi,), (si,)),
      (nk_hbm, ok_ref, (mb1,), (gk,), (sk,)),
      (na_hbm, oa_ref, (ab0, ab1), (ga0, ga1), (sa0, sa1)),
  ]
  gcp = {}
  scp = {}
  for j in range(NCHS):
    for k, (src, dst, bufs, gsems, ssems) in enumerate(pairs):
      nb = len(bufs)
      b = j % nb
      if j >= nb:
        scp[(k, (j - nb) % nb)].wait()
      gcp[k] = pltpu.async_copy(src.at[win_v.at[j]], bufs[b], gsems[b])
    for k, (src, dst, bufs, gsems, ssems) in enumerate(pairs):
      nb = len(bufs)
      b = j % nb
      gcp[k].wait()
      scp[(k, b)] = pltpu.async_copy(bufs[b], dst.at[nid_v.at[j]], ssems[b])
  scp[(0, 0)].wait()
  scp[(1, 0)].wait()
  scp[(2, 0)].wait()
  scp[(3, (NCHS - 1) % 2)].wait()
  if NCHS >= 2:
    scp[(3, (NCHS - 2) % 2)].wait()


_sc_scatter = pl.kernel(
    _sc_scatter_body,
    out_type=(),
    mesh=_sc_mesh,
    scratch_types=[
        pltpu.VMEM((NCHS, CHS), _i32),
        pltpu.VMEM((NCHS, CHS), _i32),
        pltpu.VMEM((CHS, D), _f32),
        pltpu.VMEM((CHS, M), _f32),
        pltpu.VMEM((CHS, M), _f32),
        pltpu.VMEM((CHS, 2 * M), _f32),
        pltpu.VMEM((CHS, 2 * M), _f32),
        pltpu.SemaphoreType.DMA,
        pltpu.SemaphoreType.DMA,
        pltpu.SemaphoreType.DMA,
        pltpu.SemaphoreType.DMA,
        pltpu.SemaphoreType.DMA,
        pltpu.SemaphoreType.DMA,
        pltpu.SemaphoreType.DMA,
        pltpu.SemaphoreType.DMA,
        pltpu.SemaphoreType.DMA,
        pltpu.SemaphoreType.DMA,
    ],
)


def kernel(feat, img_map, kg_map, all_map, node_id, pre_node_id,
           img_feat, kg_feat, W1, b1, W2, b2):
  nid = node_id.astype(_i32)
  pid = pre_node_id.astype(_i32)
  nid2g = nid.reshape(NW * NCH, CH)
  pid2g = pid.reshape(NW * NCH, CH)

  cur_h, cur_img, cur_kg, pre_img, pre_kg = _sc_gather(
      feat, img_map, kg_map, nid2g, pid2g)

  nid_row3 = nid.reshape(NBLK, 1, BLK)
  nid_colb = jnp.broadcast_to(nid.reshape(B, 1), (B, 128))
  b1r = b1.reshape(1, MID)
  w2r = W2.reshape(1, MID)
  b2r = jnp.broadcast_to(b2.reshape(1, 1), (1, 128))

  new_img, new_kg, new_all, new_feat, win = _dense(
      nid_row3, nid_colb, cur_h, cur_img, cur_kg, pre_img, pre_kg,
      W1, b1r, w2r, b2r, img_feat, kg_feat)

  out_feat, out_img, out_kg, out_all = _copy(feat, img_map, kg_map, all_map)

  of_r = jax.new_ref(out_feat)
  oi_r = jax.new_ref(out_img)
  ok_r = jax.new_ref(out_kg)
  oa_r = jax.new_ref(out_all)
  nid2s = nid.reshape(NW * NCHS, CHS)
  win2s = win.reshape(NW * NCHS, CHS)
  _sc_scatter(new_feat, new_img, new_kg, new_all, nid2s, win2s,
              of_r, oi_r, ok_r, oa_r)
  return of_r[...], oi_r[...], ok_r[...], oa_r[...]
